# baseline (speedup 1.0000x reference)
import functools

import jax
import jax.numpy as jnp
from jax import lax
from jax.experimental import pallas as pl
from jax.experimental.pallas import tpu as pltpu
from jax.experimental.pallas import tpu_sc as plsc

B = 4
S = 4096
D = 768
LANES = 16
NC = 2
NS = 16
NW = NC * NS
S_PER_W = S // NW
CHUNK = 32
NCHUNK = S_PER_W // CHUNK
DGRP = D // LANES


def kernel(x, token_table, pos_table):
    xr = x.astype(jnp.int32).reshape(B, NW, NCHUNK, CHUNK)
    mesh = plsc.VectorSubcoreMesh(core_axis_name="c", subcore_axis_name="s")

    @functools.partial(
        pl.kernel,
        mesh=mesh,
        out_type=jax.ShapeDtypeStruct((B * S, D), jnp.float32),
        scratch_types=[
            pltpu.VMEM((B, NCHUNK, CHUNK), jnp.int32),
            pltpu.VMEM((CHUNK, D), jnp.float32),
            pltpu.VMEM((CHUNK, D), jnp.float32),
            pltpu.VMEM((CHUNK, D), jnp.float32),
            pltpu.VMEM((CHUNK, D), jnp.float32),
            pltpu.VMEM((CHUNK, D), jnp.float32),
            pltpu.SemaphoreType.DMA,
            pltpu.SemaphoreType.DMA,
            pltpu.SemaphoreType.DMA,
        ],
    )
    def emb_sum(xr_hbm, tok_hbm, pos_hbm, out_hbm,
                idx_v, posbuf, t0, t1, t2, t3, gsem, ssem, psem):
        wid = lax.axis_index("s") * NC + lax.axis_index("c")
        sbase = wid * S_PER_W
        icps = [pltpu.async_copy(xr_hbm.at[b, wid], idx_v.at[b], psem)
                for b in range(B)]
        for cp in icps:
            cp.wait()
        bufs = (t0, t1, t2, t3)

        stores = [None] * B
        for m in range(NCHUNK):
            gathers = []
            for b in range(B):
                if stores[b] is not None:
                    stores[b].wait()
                    stores[b] = None
                gathers.append(pltpu.async_copy(
                    tok_hbm.at[idx_v.at[b, m]], bufs[b], gsem))
            pload = pltpu.async_copy(
                pos_hbm.at[pl.ds(sbase + m * CHUNK, CHUNK)], posbuf, psem)
            for cp in gathers:
                cp.wait()
            pload.wait()

            @plsc.parallel_loop(0, CHUNK, unroll=1)
            def row_add(i):
                for j in range(DGRP):
                    pv = posbuf[i, pl.ds(j * LANES, LANES)]
                    for b in range(B):
                        plsc.addupdate(
                            bufs[b].at[i, pl.ds(j * LANES, LANES)], pv)

            for b in range(B):
                stores[b] = pltpu.async_copy(
                    bufs[b],
                    out_hbm.at[pl.ds(b * S + sbase + m * CHUNK, CHUNK)],
                    ssem)
        for st in stores:
            if st is not None:
                st.wait()

    out = emb_sum(xr, token_table, pos_table)
    return out.reshape(B, S, D)


# speedup vs baseline: 1.1622x; 1.1622x over previous
import functools

import jax
import jax.numpy as jnp
from jax import lax
from jax.experimental import pallas as pl
from jax.experimental.pallas import tpu as pltpu
from jax.experimental.pallas import tpu_sc as plsc

B = 4
S = 4096
D = 768
LANES = 16
NC = 2
NS = 16
NW = NC * NS
S_PER_W = S // NW
CHUNK = 16
NCHUNK = S_PER_W // CHUNK
DGRP = D // LANES


def kernel(x, token_table, pos_table):
    xr = x.astype(jnp.int32).reshape(B, NW, NCHUNK, CHUNK)
    mesh = plsc.VectorSubcoreMesh(core_axis_name="c", subcore_axis_name="s")

    tokbuf_types = [pltpu.VMEM((CHUNK, D), jnp.float32) for _ in range(2 * B)]
    gsem_types = [pltpu.SemaphoreType.DMA for _ in range(2)]
    ssem_types = [pltpu.SemaphoreType.DMA for _ in range(2)]

    @functools.partial(
        pl.kernel,
        mesh=mesh,
        out_type=jax.ShapeDtypeStruct((B * S, D), jnp.float32),
        scratch_types=[
            pltpu.VMEM((B, NCHUNK, CHUNK), jnp.int32),
            pltpu.VMEM((CHUNK, D), jnp.float32),
            pltpu.VMEM((CHUNK, D), jnp.float32),
            *tokbuf_types,
            *gsem_types,
            *ssem_types,
            pltpu.SemaphoreType.DMA,
            pltpu.SemaphoreType.DMA,
        ],
    )
    def emb_sum(xr_hbm, tok_hbm, pos_hbm, out_hbm,
                idx_v, pos0, pos1,
                t00, t01, t02, t03, t10, t11, t12, t13,
                gsem0, gsem1, ssem0, ssem1, psem0, psem1):
        wid = lax.axis_index("s") * NC + lax.axis_index("c")
        sbase = wid * S_PER_W
        icps = [pltpu.async_copy(xr_hbm.at[b, wid], idx_v.at[b], psem0)
                for b in range(B)]
        for cp in icps:
            cp.wait()
        tokbufs = ((t00, t01, t02, t03), (t10, t11, t12, t13))
        posbufs = (pos0, pos1)
        gsems = (gsem0, gsem1)
        psems = (psem0, psem1)
        ssems = (ssem0, ssem1)

        gathers = [[None] * B, [None] * B]
        stores = [[None] * B, [None] * B]
        posloads = [None, None]

        posloads[0] = pltpu.async_copy(
            pos_hbm.at[pl.ds(sbase, CHUNK)], pos0, psem0)
        for b in range(B):
            gathers[0][b] = pltpu.async_copy(
                tok_hbm.at[idx_v.at[b, 0]], tokbufs[0][b], gsems[0])

        for m in range(NCHUNK):
            h = m % 2
            if m + 1 < NCHUNK:
                nh = (m + 1) % 2
                for b in range(B):
                    if stores[nh][b] is not None:
                        stores[nh][b].wait()
                        stores[nh][b] = None
                    gathers[nh][b] = pltpu.async_copy(
                        tok_hbm.at[idx_v.at[b, m + 1]],
                        tokbufs[nh][b], gsems[nh])
            posloads[h].wait()
            if m + 1 < NCHUNK:
                nh = (m + 1) % 2
                posloads[nh] = pltpu.async_copy(
                    pos_hbm.at[pl.ds(sbase + (m + 1) * CHUNK, CHUNK)],
                    posbufs[nh], psems[nh])
            for b in range(B):
                gathers[h][b].wait()
            bufs = tokbufs[h]
            pbuf = posbufs[h]

            @plsc.parallel_loop(0, CHUNK, unroll=1)
            def row_add(i, bufs=bufs, pbuf=pbuf):
                for j in range(DGRP):
                    pv = pbuf[i, pl.ds(j * LANES, LANES)]
                    for b in range(B):
                        plsc.addupdate(
                            bufs[b].at[i, pl.ds(j * LANES, LANES)], pv)

            for b in range(B):
                stores[h][b] = pltpu.async_copy(
                    bufs[b],
                    out_hbm.at[pl.ds(b * S + sbase + m * CHUNK, CHUNK)],
                    ssems[h])
        for half in stores:
            for st in half:
                if st is not None:
                    st.wait()

    out = emb_sum(xr, token_table, pos_table)
    return out.reshape(B, S, D)
